# baseline (speedup 1.0000x reference)
import jax
import jax.numpy as jnp
from jax import lax
from jax.experimental import pallas as pl
from jax.experimental.pallas import tpu as pltpu
from jax.experimental.pallas import tpu_sc as plsc

NUM_LOCATIONS = 1000000
NUM_TIME_SLOTS = 1440
EMBED_DIM = 64
BATCH = 16384
SEQ = 50

N = BATCH * SEQ
NC, NS = 2, 16
NW = NC * NS
PER_W = N // NW
CHUNK = 128
G = PER_W // CHUNK
D = EMBED_DIM
NBUF = 4


def _sc_body(loc_hbm, time_hbm, spat_hbm, tmp_hbm, out_hbm,
             loc_v, time_v, obuf, *sems):
    sem_g = sems[:NBUF]
    sem_s = sems[NBUF:]
    wid = lax.axis_index("s") * NC + lax.axis_index("c")
    w_base = wid * PER_W

    pltpu.sync_copy(loc_hbm.at[wid], loc_v)
    pltpu.sync_copy(time_hbm.at[wid], time_v)

    def stage_a(g, b, prime=False):
        if not prime:
            prev = w_base + (g - NBUF) * CHUNK
            pltpu.make_async_copy(
                obuf.at[b], out_hbm.at[pl.ds(prev, CHUNK)], sem_s[b]).wait()
        pltpu.async_copy(tmp_hbm.at[time_v.at[g]], obuf.at[b], sem_g[b])

    def stage_b(g, b):
        pltpu.make_async_copy(
            tmp_hbm.at[time_v.at[g]], obuf.at[b], sem_g[b]).wait()
        pltpu.async_copy(spat_hbm.at[loc_v.at[g]], obuf.at[b], sem_g[b],
                         add=True)

    def stage_c(g, b):
        pltpu.make_async_copy(
            spat_hbm.at[loc_v.at[g]], obuf.at[b], sem_g[b]).wait()
        pltpu.async_copy(
            obuf.at[b], out_hbm.at[pl.ds(w_base + g * CHUNK, CHUNK)], sem_s[b])

    stage_a(0, 0, prime=True)
    stage_a(1, 1, prime=True)
    stage_b(0, 0)

    def step(i, carry):
        for b in range(NBUF):
            g = i * NBUF + b
            stage_c(g, b)

            @pl.when(g + 1 < G)
            def _():
                stage_b(g + 1, (b + 1) % NBUF)

            @pl.when(g + 2 < G)
            def _():
                stage_a(g + 2, (b + 2) % NBUF, prime=False)
        return carry

    for b in range(NBUF):
        g = b
        stage_c(g, b)
        stage_b(g + 1, (b + 1) % NBUF)
        stage_a(g + 2, (b + 2) % NBUF, prime=(g + 2 < NBUF))

    lax.fori_loop(1, G // NBUF, step, 0)

    for b in range(NBUF):
        last = G - NBUF + b
        pltpu.make_async_copy(
            obuf.at[b], out_hbm.at[pl.ds(w_base + last * CHUNK, CHUNK)],
            sem_s[b]).wait()


@jax.jit
def _run(loc_3d, time_3d, spatial_table, temporal_table):
    mesh = plsc.VectorSubcoreMesh(core_axis_name="c", subcore_axis_name="s")
    f = pl.kernel(
        _sc_body,
        out_type=jax.ShapeDtypeStruct((N, D), jnp.float32),
        mesh=mesh,
        scratch_types=[
            pltpu.VMEM((G, CHUNK), jnp.int32),
            pltpu.VMEM((G, CHUNK), jnp.int32),
            pltpu.VMEM((NBUF, CHUNK, D), jnp.float32),
        ] + [pltpu.SemaphoreType.DMA] * (2 * NBUF),
        compiler_params=pltpu.CompilerParams(use_tc_tiling_on_sc=False),
    )
    return f(loc_3d, time_3d, spatial_table, temporal_table)


def kernel(loc_ids, time_ids, spatial_table, temporal_table):
    loc_3d = loc_ids.reshape(NW, G, CHUNK).astype(jnp.int32)
    time_3d = time_ids.reshape(NW, G, CHUNK).astype(jnp.int32)
    out = _run(loc_3d, time_3d, spatial_table, temporal_table)
    return out.reshape(BATCH, SEQ, D)


# speedup vs baseline: 1.0051x; 1.0051x over previous
import jax
import jax.numpy as jnp
from jax import lax
from jax.experimental import pallas as pl
from jax.experimental.pallas import tpu as pltpu
from jax.experimental.pallas import tpu_sc as plsc

NUM_LOCATIONS = 1000000
NUM_TIME_SLOTS = 1440
EMBED_DIM = 64
BATCH = 16384
SEQ = 50

N = BATCH * SEQ
NC, NS = 2, 16
NW = NC * NS
PER_W = N // NW
CHUNK = 128
G = PER_W // CHUNK
D = EMBED_DIM
SEQ_PAD = 56
WIDE = 128
N_PAD = BATCH * SEQ_PAD
NBUF = 4


def _sc_body(loc_hbm, time_hbm, ridx_hbm, spat_hbm, tmp_hbm, out_hbm,
             loc_i, time_i, ridx_i, gbuf, obuf, *sems):
    sem_i = sems[:NBUF]
    sem_g = sems[NBUF:2 * NBUF]
    sem_s = sems[2 * NBUF:]
    wid = lax.axis_index("s") * NC + lax.axis_index("c")

    def stage_p(g, b, prime=False):
        if not prime:
            pltpu.make_async_copy(
                obuf.at[b], out_hbm.at[ridx_i.at[b]], sem_s[b]).wait()
        pltpu.async_copy(loc_hbm.at[wid, g], loc_i.at[b], sem_i[b])
        pltpu.async_copy(time_hbm.at[wid, g], time_i.at[b], sem_i[b])

    def stage_a(g, b):
        pltpu.make_async_copy(loc_hbm.at[wid, g], loc_i.at[b], sem_i[b]).wait()
        pltpu.make_async_copy(time_hbm.at[wid, g], time_i.at[b], sem_i[b]).wait()
        pltpu.async_copy(tmp_hbm.at[time_i.at[b]], gbuf.at[b], sem_g[b])

    def stage_b(g, b):
        pltpu.make_async_copy(
            tmp_hbm.at[time_i.at[b]], gbuf.at[b], sem_g[b]).wait()
        pltpu.async_copy(spat_hbm.at[loc_i.at[b]], gbuf.at[b], sem_g[b],
                         add=True)
        pltpu.async_copy(ridx_hbm.at[wid, g], ridx_i.at[b], sem_g[b])

    def stage_c(g, b):
        pltpu.make_async_copy(
            spat_hbm.at[loc_i.at[b]], gbuf.at[b], sem_g[b]).wait()
        pltpu.make_async_copy(
            ridx_hbm.at[wid, g], ridx_i.at[b], sem_g[b]).wait()

        def widen_row(r, c):
            for j in range(D // 16):
                sl = pl.ds(16 * j, 16)
                obuf[b, r, sl] = gbuf[b, r, sl]
            return c

        lax.fori_loop(0, CHUNK, widen_row, 0)
        pltpu.async_copy(obuf.at[b], out_hbm.at[ridx_i.at[b]], sem_s[b])

    stage_p(0, 0, prime=True)
    stage_p(1, 1, prime=True)
    stage_p(2, 2, prime=True)
    stage_a(0, 0)
    stage_a(1, 1)
    stage_b(0, 0)

    for b in range(NBUF):
        g = b
        stage_c(g, b)
        stage_b(g + 1, (b + 1) % NBUF)
        stage_a(g + 2, (b + 2) % NBUF)
        stage_p(g + 3, (b + 3) % NBUF, prime=(g + 3 < NBUF))

    def step(i, carry):
        for b in range(NBUF):
            g = i * NBUF + b
            stage_c(g, b)

            @pl.when(g + 1 < G)
            def _():
                stage_b(g + 1, (b + 1) % NBUF)

            @pl.when(g + 2 < G)
            def _():
                stage_a(g + 2, (b + 2) % NBUF)

            @pl.when(g + 3 < G)
            def _():
                stage_p(g + 3, (b + 3) % NBUF, prime=False)
        return carry

    lax.fori_loop(1, G // NBUF, step, 0)

    for b in range(NBUF):
        pltpu.make_async_copy(
            obuf.at[b], out_hbm.at[ridx_i.at[b]], sem_s[b]).wait()


@jax.jit
def _run(loc_3d, time_3d, ridx_3d, spatial_table, temporal_table):
    mesh = plsc.VectorSubcoreMesh(core_axis_name="c", subcore_axis_name="s")
    f = pl.kernel(
        _sc_body,
        out_type=jax.ShapeDtypeStruct((N_PAD, WIDE), jnp.float32),
        mesh=mesh,
        scratch_types=[
            pltpu.VMEM((NBUF, CHUNK), jnp.int32),
            pltpu.VMEM((NBUF, CHUNK), jnp.int32),
            pltpu.VMEM((NBUF, CHUNK), jnp.int32),
            pltpu.VMEM((NBUF, CHUNK, D), jnp.float32),
            pltpu.VMEM((NBUF, CHUNK, WIDE), jnp.float32),
        ] + [pltpu.SemaphoreType.DMA] * (3 * NBUF),
        compiler_params=pltpu.CompilerParams(use_tc_tiling_on_sc=False),
    )
    return f(loc_3d, time_3d, ridx_3d, spatial_table, temporal_table)


def kernel(loc_ids, time_ids, spatial_table, temporal_table):
    loc_3d = loc_ids.reshape(NW, G, CHUNK).astype(jnp.int32)
    time_3d = time_ids.reshape(NW, G, CHUNK).astype(jnp.int32)
    n = jnp.arange(N, dtype=jnp.int32)
    ridx_3d = (n + (n // SEQ) * (SEQ_PAD - SEQ)).reshape(NW, G, CHUNK)
    out_wide = _run(loc_3d, time_3d, ridx_3d, spatial_table, temporal_table)
    return out_wide.reshape(BATCH, SEQ_PAD, WIDE)[:, :SEQ, :D]


# speedup vs baseline: 1.2958x; 1.2892x over previous
import jax
import jax.numpy as jnp
from jax import lax
from jax.experimental import pallas as pl
from jax.experimental.pallas import tpu as pltpu
from jax.experimental.pallas import tpu_sc as plsc

NUM_LOCATIONS = 1000000
NUM_TIME_SLOTS = 1440
EMBED_DIM = 64
BATCH = 16384
SEQ = 50

N = BATCH * SEQ
NC, NS = 2, 16
NW = NC * NS
PER_W = N // NW
CHUNK = 128
G = PER_W // CHUNK
D = EMBED_DIM
SEQ_PAD = 56
WIDE = 128
N_PAD = BATCH * SEQ_PAD
NBUF = 4


def _sc_body(loc_hbm, time_hbm, ridx_hbm, spat_hbm, tmp_hbm, out_hbm,
             loc_v, time_v, ridx_v, gbuf, *sems):
    sem_g = sems[:NBUF]
    sem_s = sems[NBUF:]
    wid = lax.axis_index("s") * NC + lax.axis_index("c")

    pltpu.sync_copy(loc_hbm.at[wid], loc_v)
    pltpu.sync_copy(time_hbm.at[wid], time_v)
    pltpu.sync_copy(ridx_hbm.at[wid], ridx_v)

    def stage_a(g, b, prime=False):
        if not prime:
            pltpu.make_async_copy(
                gbuf.at[b], out_hbm.at[ridx_v.at[g - NBUF]], sem_s[b]).wait()
        pltpu.async_copy(tmp_hbm.at[time_v.at[g]], gbuf.at[b], sem_g[b])

    def stage_b(g, b):
        pltpu.make_async_copy(
            tmp_hbm.at[time_v.at[g]], gbuf.at[b], sem_g[b]).wait()
        pltpu.async_copy(spat_hbm.at[loc_v.at[g]], gbuf.at[b], sem_g[b],
                         add=True)

    def stage_c(g, b):
        pltpu.make_async_copy(
            spat_hbm.at[loc_v.at[g]], gbuf.at[b], sem_g[b]).wait()
        pltpu.async_copy(gbuf.at[b], out_hbm.at[ridx_v.at[g]], sem_s[b])

    stage_a(0, 0, prime=True)
    stage_a(1, 1, prime=True)
    stage_b(0, 0)

    for b in range(NBUF):
        g = b
        stage_c(g, b)
        stage_b(g + 1, (b + 1) % NBUF)
        stage_a(g + 2, (b + 2) % NBUF, prime=(g + 2 < NBUF))

    def step(i, carry):
        for b in range(NBUF):
            g = i * NBUF + b
            stage_c(g, b)

            @pl.when(g + 1 < G)
            def _():
                stage_b(g + 1, (b + 1) % NBUF)

            @pl.when(g + 2 < G)
            def _():
                stage_a(g + 2, (b + 2) % NBUF, prime=False)
        return carry

    lax.fori_loop(1, G // NBUF, step, 0)

    for b in range(NBUF):
        last = G - NBUF + b
        pltpu.make_async_copy(
            gbuf.at[b], out_hbm.at[ridx_v.at[last]], sem_s[b]).wait()


@jax.jit
def _run(loc_3d, time_3d, ridx_3d, spatial_table, temporal_table):
    mesh = plsc.VectorSubcoreMesh(core_axis_name="c", subcore_axis_name="s")
    f = pl.kernel(
        _sc_body,
        out_type=jax.ShapeDtypeStruct((2 * N_PAD, D), jnp.float32),
        mesh=mesh,
        scratch_types=[
            pltpu.VMEM((G, CHUNK), jnp.int32),
            pltpu.VMEM((G, CHUNK), jnp.int32),
            pltpu.VMEM((G, CHUNK), jnp.int32),
            pltpu.VMEM((NBUF, CHUNK, D), jnp.float32),
        ] + [pltpu.SemaphoreType.DMA] * (2 * NBUF),
        compiler_params=pltpu.CompilerParams(use_tc_tiling_on_sc=False),
    )
    return f(loc_3d, time_3d, ridx_3d, spatial_table, temporal_table)


def kernel(loc_ids, time_ids, spatial_table, temporal_table):
    loc_3d = loc_ids.reshape(NW, G, CHUNK).astype(jnp.int32)
    time_3d = time_ids.reshape(NW, G, CHUNK).astype(jnp.int32)
    n = jnp.arange(N, dtype=jnp.int32)
    ridx_3d = (2 * (n + (n // SEQ) * (SEQ_PAD - SEQ))).reshape(NW, G, CHUNK)
    out_half = _run(loc_3d, time_3d, ridx_3d, spatial_table, temporal_table)
    return out_half.reshape(BATCH, SEQ_PAD, WIDE)[:, :SEQ, :D]


# speedup vs baseline: 1.3775x; 1.0630x over previous
import jax
import jax.numpy as jnp
from jax import lax
from jax.experimental import pallas as pl
from jax.experimental.pallas import tpu as pltpu
from jax.experimental.pallas import tpu_sc as plsc

NUM_LOCATIONS = 1000000
NUM_TIME_SLOTS = 1440
EMBED_DIM = 64
BATCH = 16384
SEQ = 50

N = BATCH * SEQ
NC, NS = 2, 16
NW = NC * NS
PER_W = N // NW
CHUNK = 128
G = PER_W // CHUNK
D = EMBED_DIM
SEQ_PAD = 56
WIDE = 128
N_PAD = BATCH * SEQ_PAD
NBUF = 4


def _sc_body(loc_hbm, time_hbm, ridx_hbm, spat_hbm, tmp_hbm, out_hbm,
             loc_v, time_v, ridx_v, gbuf, tmp_sh, *sems):
    sem_g = sems[:NBUF]
    sem_s = sems[NBUF:]
    wid = lax.axis_index("s") * NC + lax.axis_index("c")

    @pl.when(lax.axis_index("s") == 0)
    def _():
        pltpu.sync_copy(tmp_hbm, tmp_sh)

    pltpu.sync_copy(loc_hbm.at[wid], loc_v)
    pltpu.sync_copy(time_hbm.at[wid], time_v)
    pltpu.sync_copy(ridx_hbm.at[wid], ridx_v)
    plsc.subcore_barrier()

    def stage_a(g, b, prime=False):
        if not prime:
            pltpu.make_async_copy(
                gbuf.at[b], out_hbm.at[ridx_v.at[g - NBUF]], sem_s[b]).wait()
        pltpu.async_copy(tmp_sh.at[time_v.at[g]], gbuf.at[b], sem_g[b])

    def stage_b(g, b):
        pltpu.make_async_copy(
            tmp_sh.at[time_v.at[g]], gbuf.at[b], sem_g[b]).wait()
        pltpu.async_copy(spat_hbm.at[loc_v.at[g]], gbuf.at[b], sem_g[b],
                         add=True)

    def stage_c(g, b):
        pltpu.make_async_copy(
            spat_hbm.at[loc_v.at[g]], gbuf.at[b], sem_g[b]).wait()
        pltpu.async_copy(gbuf.at[b], out_hbm.at[ridx_v.at[g]], sem_s[b])

    stage_a(0, 0, prime=True)
    stage_a(1, 1, prime=True)
    stage_b(0, 0)

    for b in range(NBUF):
        g = b
        stage_c(g, b)
        stage_b(g + 1, (b + 1) % NBUF)
        stage_a(g + 2, (b + 2) % NBUF, prime=(g + 2 < NBUF))

    def step(i, carry):
        for b in range(NBUF):
            g = i * NBUF + b
            stage_c(g, b)

            @pl.when(g + 1 < G)
            def _():
                stage_b(g + 1, (b + 1) % NBUF)

            @pl.when(g + 2 < G)
            def _():
                stage_a(g + 2, (b + 2) % NBUF, prime=False)
        return carry

    lax.fori_loop(1, G // NBUF, step, 0)

    for b in range(NBUF):
        last = G - NBUF + b
        pltpu.make_async_copy(
            gbuf.at[b], out_hbm.at[ridx_v.at[last]], sem_s[b]).wait()


@jax.jit
def _run(loc_3d, time_3d, ridx_3d, spatial_table, temporal_table):
    mesh = plsc.VectorSubcoreMesh(core_axis_name="c", subcore_axis_name="s")
    f = pl.kernel(
        _sc_body,
        out_type=jax.ShapeDtypeStruct((2 * N_PAD, D), jnp.float32),
        mesh=mesh,
        scratch_types=[
            pltpu.VMEM((G, CHUNK), jnp.int32),
            pltpu.VMEM((G, CHUNK), jnp.int32),
            pltpu.VMEM((G, CHUNK), jnp.int32),
            pltpu.VMEM((NBUF, CHUNK, D), jnp.float32),
            pltpu.VMEM_SHARED((NUM_TIME_SLOTS, D), jnp.float32),
        ] + [pltpu.SemaphoreType.DMA] * (2 * NBUF),
        compiler_params=pltpu.CompilerParams(use_tc_tiling_on_sc=False),
    )
    return f(loc_3d, time_3d, ridx_3d, spatial_table, temporal_table)


def kernel(loc_ids, time_ids, spatial_table, temporal_table):
    loc_3d = loc_ids.reshape(NW, G, CHUNK).astype(jnp.int32)
    time_3d = time_ids.reshape(NW, G, CHUNK).astype(jnp.int32)
    n = jnp.arange(N, dtype=jnp.int32)
    ridx_3d = (2 * (n + (n // SEQ) * (SEQ_PAD - SEQ))).reshape(NW, G, CHUNK)
    spat_packed = lax.optimization_barrier(
        spatial_table.reshape(NUM_LOCATIONS // 2, 2 * D))
    spat_lin = spat_packed.reshape(NUM_LOCATIONS, D)
    out_half = _run(loc_3d, time_3d, ridx_3d, spat_lin, temporal_table)
    return out_half.reshape(BATCH, SEQ_PAD, WIDE)[:, :SEQ, :D]


# speedup vs baseline: 1.3791x; 1.0012x over previous
import jax
import jax.numpy as jnp
from jax import lax
from jax.experimental import pallas as pl
from jax.experimental.pallas import tpu as pltpu
from jax.experimental.pallas import tpu_sc as plsc

NUM_LOCATIONS = 1000000
NUM_TIME_SLOTS = 1440
EMBED_DIM = 64
BATCH = 16384
SEQ = 50

N = BATCH * SEQ
NC, NS = 2, 16
NW = NC * NS
PER_W = N // NW
CHUNK = 128
G = PER_W // CHUNK
D = EMBED_DIM
SEQ_PAD = 56
WIDE = 128
N_PAD = BATCH * SEQ_PAD
NBUF = 4


def _sc_body(loc_hbm, time_hbm, ridx_hbm, spat_hbm, tmp_hbm, out_hbm,
             loc_v, time_v, ridx_v, gbuf, tmp_sh, *sems):
    sem_g = sems[:NBUF]
    sem_s = sems[NBUF:]
    wid = lax.axis_index("s") * NC + lax.axis_index("c")

    @pl.when(lax.axis_index("s") == 0)
    def _():
        pltpu.sync_copy(tmp_hbm, tmp_sh)

    pltpu.sync_copy(loc_hbm.at[wid], loc_v)
    pltpu.sync_copy(time_hbm.at[wid], time_v)
    pltpu.sync_copy(ridx_hbm.at[wid], ridx_v)
    plsc.subcore_barrier()

    def stage_a(g, b, prime=False):
        if not prime:
            pltpu.make_async_copy(
                gbuf.at[b], out_hbm.at[ridx_v.at[g - NBUF]], sem_s[b]).wait()
        pltpu.async_copy(tmp_sh.at[time_v.at[g]], gbuf.at[b], sem_g[b])

    def stage_b(g, b):
        pltpu.make_async_copy(
            tmp_sh.at[time_v.at[g]], gbuf.at[b], sem_g[b]).wait()
        pltpu.async_copy(spat_hbm.at[loc_v.at[g]], gbuf.at[b], sem_g[b],
                         add=True)

    def stage_c(g, b):
        pltpu.make_async_copy(
            spat_hbm.at[loc_v.at[g]], gbuf.at[b], sem_g[b]).wait()
        pltpu.async_copy(gbuf.at[b], out_hbm.at[ridx_v.at[g]], sem_s[b])

    stage_a(0, 0, prime=True)
    stage_a(1, 1, prime=True)
    stage_b(0, 0)

    for b in range(NBUF):
        g = b
        stage_c(g, b)
        stage_b(g + 1, (b + 1) % NBUF)
        stage_a(g + 2, (b + 2) % NBUF, prime=(g + 2 < NBUF))

    def step(i, carry):
        for b in range(NBUF):
            g = i * NBUF + b
            stage_c(g, b)

            @pl.when(g + 1 < G)
            def _():
                stage_b(g + 1, (b + 1) % NBUF)

            @pl.when(g + 2 < G)
            def _():
                stage_a(g + 2, (b + 2) % NBUF, prime=False)
        return carry

    lax.fori_loop(1, G // NBUF, step, 0)

    for b in range(NBUF):
        last = G - NBUF + b
        pltpu.make_async_copy(
            gbuf.at[b], out_hbm.at[ridx_v.at[last]], sem_s[b]).wait()


@jax.jit
def _run(loc_3d, time_3d, ridx_3d, spatial_table, temporal_table):
    mesh = plsc.VectorSubcoreMesh(core_axis_name="c", subcore_axis_name="s")
    f = pl.kernel(
        _sc_body,
        out_type=jax.ShapeDtypeStruct((2 * N_PAD, D), jnp.float32),
        mesh=mesh,
        scratch_types=[
            pltpu.VMEM((G, CHUNK), jnp.int32),
            pltpu.VMEM((G, CHUNK), jnp.int32),
            pltpu.VMEM((G, CHUNK), jnp.int32),
            pltpu.VMEM((NBUF, CHUNK, D), jnp.float32),
            pltpu.VMEM_SHARED((NUM_TIME_SLOTS, D), jnp.float32),
        ] + [pltpu.SemaphoreType.DMA] * (2 * NBUF),
        compiler_params=pltpu.CompilerParams(use_tc_tiling_on_sc=False),
    )
    return f(loc_3d, time_3d, ridx_3d, spatial_table, temporal_table)


def kernel(loc_ids, time_ids, spatial_table, temporal_table):
    loc_3d = loc_ids.reshape(NW, G, CHUNK).astype(jnp.int32)
    time_3d = time_ids.reshape(NW, G, CHUNK).astype(jnp.int32)
    n = jnp.arange(N, dtype=jnp.int32)
    ridx_3d = (2 * (n + (n // SEQ) * (SEQ_PAD - SEQ))).reshape(NW, G, CHUNK)
    out_half = _run(loc_3d, time_3d, ridx_3d, spatial_table, temporal_table)
    return out_half.reshape(BATCH, SEQ_PAD, WIDE)[:, :SEQ, :D]
